# Initial kernel scaffold; baseline (speedup 1.0000x reference)
import jax
import jax.numpy as jnp
from jax.experimental import pallas as pl


def kernel(X, Wg1, bg1, Wg2, bg2, Wp1, bp1, Ws1, Wn1, bc1, Wp2, bp2, Ws2, Wn2, bc2, Wf1, bf1, Wf2, bf2, Wf3, bf3):
    raise NotImplementedError("write your pallas kernel here")


# speedup vs baseline: 224.3947x; 224.3947x over previous
import jax
import jax.numpy as jnp
from jax.experimental import pallas as pl

_B, _NOBS, _H, _NPG = 2048, 15, 64, 16
_BB = 256


def _fused_kernel(x_ref, nd_ref,
                  wg1_ref, bg1_ref, wg2_ref, bg2_ref,
                  wp1_ref, bp1_ref, ws1_ref, wn1_ref, bc1_ref,
                  wp2_ref, bp2_ref, ws2_ref, wn2_ref, bc2_ref,
                  wf1g_ref, wf1o_ref, bf1_ref, wf2_ref, bf2_ref,
                  wf3_ref, bf3_ref, out_ref):
    n = _NPG
    bb = _BB
    nd = nd_ref[:]

    m1 = jnp.maximum(jnp.dot(nd, wp1_ref[:], preferred_element_type=jnp.float32)
                     + bp1_ref[:], 0.0)
    neigh1 = jnp.max(m1.reshape(bb, n, 4), axis=1)
    n1w = jnp.dot(neigh1, wn1_ref[:], preferred_element_type=jnp.float32)
    n1w_full = jnp.broadcast_to(n1w[:, None, :], (bb, n, _H)).reshape(bb * n, _H)
    h1 = jnp.maximum(jnp.dot(nd, ws1_ref[:], preferred_element_type=jnp.float32)
                     + n1w_full + bc1_ref[:], 0.0)

    m2 = jnp.maximum(jnp.dot(h1, wp2_ref[:], preferred_element_type=jnp.float32)
                     + bp2_ref[:], 0.0)
    neigh2 = jnp.max(m2.reshape(bb, n, _H), axis=1)
    h1mean = jnp.mean(h1.reshape(bb, n, _H), axis=1)
    obs = (jnp.dot(h1mean, ws2_ref[:], preferred_element_type=jnp.float32)
           + jnp.dot(neigh2, wn2_ref[:], preferred_element_type=jnp.float32)
           + bc2_ref[:])

    xg = x_ref[:, :16]
    g = jnp.maximum(jnp.dot(xg, wg1_ref[:], preferred_element_type=jnp.float32)
                    + bg1_ref[:], 0.0)
    g = jnp.dot(g, wg2_ref[:], preferred_element_type=jnp.float32) + bg2_ref[:]

    c = jnp.maximum(jnp.dot(g, wf1g_ref[:], preferred_element_type=jnp.float32)
                    + jnp.dot(obs, wf1o_ref[:], preferred_element_type=jnp.float32)
                    + bf1_ref[:], 0.0)
    c = jnp.maximum(jnp.dot(c, wf2_ref[:], preferred_element_type=jnp.float32)
                    + bf2_ref[:], 0.0)
    out_ref[:] = jnp.tanh(jnp.dot(c, wf3_ref[:], preferred_element_type=jnp.float32)
                          + bf3_ref[:])


def kernel(X, Wg1, bg1, Wg2, bg2, Wp1, bp1, Ws1, Wn1, bc1, Wp2, bp2, Ws2, Wn2,
           bc2, Wf1, bf1, Wf2, bf2, Wf3, bf3):
    batch = X.shape[0]
    n = _NPG
    obst = X[:, 16:].reshape(batch, _NOBS, 4)
    sentinel = jnp.zeros((batch, 1, 4), X.dtype)
    nodes = jnp.concatenate([sentinel, obst], axis=1).reshape(batch * n, 4)

    grid = (batch // _BB,)
    full = lambda *s: pl.BlockSpec(s, lambda i: (0,) * len(s))
    row2 = lambda b: (1, b)

    specs = [
        pl.BlockSpec((_BB, X.shape[1]), lambda i: (i, 0)),
        pl.BlockSpec((_BB * n, 4), lambda i: (i, 0)),
        full(16, _H), full(*row2(_H)),
        full(_H, _H), full(*row2(_H)),
        full(4, 4), full(*row2(4)),
        full(4, _H), full(4, _H), full(*row2(_H)),
        full(_H, _H), full(*row2(_H)),
        full(_H, _H), full(_H, _H), full(*row2(_H)),
        full(_H, _H), full(_H, _H), full(*row2(_H)),
        full(_H, _H), full(*row2(_H)),
        full(_H, 8), full(*row2(8)),
    ]
    out = pl.pallas_call(
        _fused_kernel,
        grid=grid,
        in_specs=specs,
        out_specs=pl.BlockSpec((_BB, 8), lambda i: (i, 0)),
        out_shape=jax.ShapeDtypeStruct((batch, 8), jnp.float32),
    )(X, nodes,
      Wg1, bg1.reshape(1, -1), Wg2, bg2.reshape(1, -1),
      Wp1, bp1.reshape(1, -1), Ws1, Wn1, bc1.reshape(1, -1),
      Wp2, bp2.reshape(1, -1), Ws2, Wn2, bc2.reshape(1, -1),
      Wf1[:_H], Wf1[_H:], bf1.reshape(1, -1), Wf2, bf2.reshape(1, -1),
      Wf3, bf3.reshape(1, -1))
    return out


# speedup vs baseline: 440.6634x; 1.9638x over previous
import jax
import jax.numpy as jnp
from jax.experimental import pallas as pl

_NOBS, _H, _NPG = 15, 64, 16
_BB = 256


def _fold_max(a, bb):
    k = a.shape[0] // bb
    while k > 1:
        half = (k // 2) * bb
        a = jnp.maximum(a[:half], a[half:])
        k //= 2
    return a


def _fold_sum(a, bb):
    k = a.shape[0] // bb
    while k > 1:
        half = (k // 2) * bb
        a = a[:half] + a[half:]
        k //= 2
    return a


def _fused_kernel(x_ref,
                  wg1_ref, bg1_ref, wg2_ref, bg2_ref,
                  wp1_ref, bp1_ref, ws1_ref, wn1_ref, bc1_ref,
                  wp2_ref, bp2_ref, ws2_ref, wn2_ref, bc2_ref,
                  wf1g_ref, wf1o_ref, bf1_ref, wf2_ref, bf2_ref,
                  wf3_ref, bf3_ref, out_ref):
    n = _NPG
    bb = x_ref.shape[0]
    x = x_ref[:]

    nd = jnp.concatenate(
        [jnp.zeros((bb, 4), jnp.float32)]
        + [x[:, 12 + 4 * j:16 + 4 * j] for j in range(1, n)], axis=0)

    m1 = jnp.maximum(jnp.dot(nd, wp1_ref[:], preferred_element_type=jnp.float32)
                     + bp1_ref[:], 0.0)
    neigh1 = _fold_max(m1, bb)
    n1w = jnp.dot(neigh1, wn1_ref[:], preferred_element_type=jnp.float32)
    h1 = jnp.maximum(jnp.dot(nd, ws1_ref[:], preferred_element_type=jnp.float32)
                     + jnp.tile(n1w, (n, 1)) + bc1_ref[:], 0.0)

    m2 = jnp.maximum(jnp.dot(h1, wp2_ref[:], preferred_element_type=jnp.float32)
                     + bp2_ref[:], 0.0)
    neigh2 = _fold_max(m2, bb)
    h1mean = _fold_sum(h1, bb) * (1.0 / n)
    obs = (jnp.dot(h1mean, ws2_ref[:], preferred_element_type=jnp.float32)
           + jnp.dot(neigh2, wn2_ref[:], preferred_element_type=jnp.float32)
           + bc2_ref[:])

    g = jnp.maximum(jnp.dot(x[:, :16], wg1_ref[:], preferred_element_type=jnp.float32)
                    + bg1_ref[:], 0.0)
    g = jnp.dot(g, wg2_ref[:], preferred_element_type=jnp.float32) + bg2_ref[:]

    c = jnp.maximum(jnp.dot(g, wf1g_ref[:], preferred_element_type=jnp.float32)
                    + jnp.dot(obs, wf1o_ref[:], preferred_element_type=jnp.float32)
                    + bf1_ref[:], 0.0)
    c = jnp.maximum(jnp.dot(c, wf2_ref[:], preferred_element_type=jnp.float32)
                    + bf2_ref[:], 0.0)
    out_ref[:] = jnp.tanh(jnp.dot(c, wf3_ref[:], preferred_element_type=jnp.float32)
                          + bf3_ref[:])


def kernel(X, Wg1, bg1, Wg2, bg2, Wp1, bp1, Ws1, Wn1, bc1, Wp2, bp2, Ws2, Wn2,
           bc2, Wf1, bf1, Wf2, bf2, Wf3, bf3):
    batch = X.shape[0]

    grid = (batch // _BB,)
    full = lambda *s: pl.BlockSpec(s, lambda i: (0,) * len(s))

    specs = [
        pl.BlockSpec((_BB, X.shape[1]), lambda i: (i, 0)),
        full(16, _H), full(1, _H),
        full(_H, _H), full(1, _H),
        full(4, 4), full(1, 4),
        full(4, _H), full(4, _H), full(1, _H),
        full(_H, _H), full(1, _H),
        full(_H, _H), full(_H, _H), full(1, _H),
        full(_H, _H), full(_H, _H), full(1, _H),
        full(_H, _H), full(1, _H),
        full(_H, 8), full(1, 8),
    ]
    out = pl.pallas_call(
        _fused_kernel,
        grid=grid,
        in_specs=specs,
        out_specs=pl.BlockSpec((_BB, 8), lambda i: (i, 0)),
        out_shape=jax.ShapeDtypeStruct((batch, 8), jnp.float32),
    )(X,
      Wg1, bg1.reshape(1, -1), Wg2, bg2.reshape(1, -1),
      Wp1, bp1.reshape(1, -1), Ws1, Wn1, bc1.reshape(1, -1),
      Wp2, bp2.reshape(1, -1), Ws2, Wn2, bc2.reshape(1, -1),
      Wf1[:_H], Wf1[_H:], bf1.reshape(1, -1), Wf2, bf2.reshape(1, -1),
      Wf3, bf3.reshape(1, -1))
    return out


# speedup vs baseline: 517.5015x; 1.1744x over previous
import jax
import jax.numpy as jnp
from jax.experimental import pallas as pl

_NOBS, _H, _NPG = 15, 64, 16
_BB = 512


def _fold_max(a, bb):
    k = a.shape[0] // bb
    while k > 1:
        half = (k // 2) * bb
        a = jnp.maximum(a[:half], a[half:])
        k //= 2
    return a


def _fold_sum(a, bb):
    k = a.shape[0] // bb
    while k > 1:
        half = (k // 2) * bb
        a = a[:half] + a[half:]
        k //= 2
    return a


def _fused_kernel(x_ref,
                  wg1_ref, bg1_ref, wg2_ref, bg2_ref,
                  wp1_ref, bp1_ref, ws1_ref, wn1_ref, bc1_ref,
                  wp2_ref, bp2_ref, ws2_ref, wn2_ref, bc2_ref,
                  wf1g_ref, wf1o_ref, bf1_ref, wf2_ref, bf2_ref,
                  wf3_ref, bf3_ref, out_ref):
    n = _NPG
    bb = x_ref.shape[0]
    x = x_ref[:]

    nd = jnp.concatenate(
        [jnp.zeros((bb, 4), jnp.float32)]
        + [x[:, 12 + 4 * j:16 + 4 * j] for j in range(1, n)], axis=0)

    m1 = jnp.maximum(jnp.dot(nd, wp1_ref[:], preferred_element_type=jnp.float32)
                     + bp1_ref[:], 0.0)
    neigh1 = _fold_max(m1, bb)
    n1w = jnp.dot(neigh1, wn1_ref[:], preferred_element_type=jnp.float32)
    h1 = jnp.maximum(jnp.dot(nd, ws1_ref[:], preferred_element_type=jnp.float32)
                     + jnp.tile(n1w, (n, 1)) + bc1_ref[:], 0.0)

    m2 = jnp.maximum(jnp.dot(h1, wp2_ref[:], preferred_element_type=jnp.float32)
                     + bp2_ref[:], 0.0)
    neigh2 = _fold_max(m2, bb)
    h1mean = _fold_sum(h1, bb) * (1.0 / n)
    obs = (jnp.dot(h1mean, ws2_ref[:], preferred_element_type=jnp.float32)
           + jnp.dot(neigh2, wn2_ref[:], preferred_element_type=jnp.float32)
           + bc2_ref[:])

    g = jnp.maximum(jnp.dot(x[:, :16], wg1_ref[:], preferred_element_type=jnp.float32)
                    + bg1_ref[:], 0.0)
    g = jnp.dot(g, wg2_ref[:], preferred_element_type=jnp.float32) + bg2_ref[:]

    c = jnp.maximum(jnp.dot(g, wf1g_ref[:], preferred_element_type=jnp.float32)
                    + jnp.dot(obs, wf1o_ref[:], preferred_element_type=jnp.float32)
                    + bf1_ref[:], 0.0)
    c = jnp.maximum(jnp.dot(c, wf2_ref[:], preferred_element_type=jnp.float32)
                    + bf2_ref[:], 0.0)
    out_ref[:] = jnp.tanh(jnp.dot(c, wf3_ref[:], preferred_element_type=jnp.float32)
                          + bf3_ref[:])


def kernel(X, Wg1, bg1, Wg2, bg2, Wp1, bp1, Ws1, Wn1, bc1, Wp2, bp2, Ws2, Wn2,
           bc2, Wf1, bf1, Wf2, bf2, Wf3, bf3):
    batch = X.shape[0]

    grid = (batch // _BB,)
    full = lambda *s: pl.BlockSpec(s, lambda i: (0,) * len(s))

    specs = [
        pl.BlockSpec((_BB, X.shape[1]), lambda i: (i, 0)),
        full(16, _H), full(1, _H),
        full(_H, _H), full(1, _H),
        full(4, 4), full(1, 4),
        full(4, _H), full(4, _H), full(1, _H),
        full(_H, _H), full(1, _H),
        full(_H, _H), full(_H, _H), full(1, _H),
        full(_H, _H), full(_H, _H), full(1, _H),
        full(_H, _H), full(1, _H),
        full(_H, 8), full(1, 8),
    ]
    out = pl.pallas_call(
        _fused_kernel,
        grid=grid,
        in_specs=specs,
        out_specs=pl.BlockSpec((_BB, 8), lambda i: (i, 0)),
        out_shape=jax.ShapeDtypeStruct((batch, 8), jnp.float32),
    )(X,
      Wg1, bg1.reshape(1, -1), Wg2, bg2.reshape(1, -1),
      Wp1, bp1.reshape(1, -1), Ws1, Wn1, bc1.reshape(1, -1),
      Wp2, bp2.reshape(1, -1), Ws2, Wn2, bc2.reshape(1, -1),
      Wf1[:_H], Wf1[_H:], bf1.reshape(1, -1), Wf2, bf2.reshape(1, -1),
      Wf3, bf3.reshape(1, -1))
    return out


# speedup vs baseline: 550.5044x; 1.0638x over previous
import jax
import jax.numpy as jnp
from jax.experimental import pallas as pl

_NOBS, _H, _NPG = 15, 64, 16
_BB = 1024


def _fold_max(a, bb):
    k = a.shape[0] // bb
    while k > 1:
        half = (k // 2) * bb
        a = jnp.maximum(a[:half], a[half:])
        k //= 2
    return a


def _fold_sum(a, bb):
    k = a.shape[0] // bb
    while k > 1:
        half = (k // 2) * bb
        a = a[:half] + a[half:]
        k //= 2
    return a


def _fused_kernel(x_ref,
                  wg1_ref, bg1_ref, wg2_ref, bg2_ref,
                  wp1_ref, bp1_ref, ws1_ref, wn1_ref, bc1_ref,
                  wp2_ref, bp2_ref, ws2_ref, wn2_ref, bc2_ref,
                  wf1g_ref, wf1o_ref, bf1_ref, wf2_ref, bf2_ref,
                  wf3_ref, bf3_ref, out_ref):
    n = _NPG
    bb = x_ref.shape[0]
    x = x_ref[:]

    nd = jnp.concatenate(
        [jnp.zeros((bb, 4), jnp.float32)]
        + [x[:, 12 + 4 * j:16 + 4 * j] for j in range(1, n)], axis=0)

    m1 = jnp.maximum(jnp.dot(nd, wp1_ref[:], preferred_element_type=jnp.float32)
                     + bp1_ref[:], 0.0)
    neigh1 = _fold_max(m1, bb)
    n1w = jnp.dot(neigh1, wn1_ref[:], preferred_element_type=jnp.float32)
    h1 = jnp.maximum(jnp.dot(nd, ws1_ref[:], preferred_element_type=jnp.float32)
                     + jnp.tile(n1w, (n, 1)) + bc1_ref[:], 0.0)

    m2 = jnp.maximum(jnp.dot(h1, wp2_ref[:], preferred_element_type=jnp.float32)
                     + bp2_ref[:], 0.0)
    neigh2 = _fold_max(m2, bb)
    h1mean = _fold_sum(h1, bb) * (1.0 / n)
    obs = (jnp.dot(h1mean, ws2_ref[:], preferred_element_type=jnp.float32)
           + jnp.dot(neigh2, wn2_ref[:], preferred_element_type=jnp.float32)
           + bc2_ref[:])

    g = jnp.maximum(jnp.dot(x[:, :16], wg1_ref[:], preferred_element_type=jnp.float32)
                    + bg1_ref[:], 0.0)
    g = jnp.dot(g, wg2_ref[:], preferred_element_type=jnp.float32) + bg2_ref[:]

    c = jnp.maximum(jnp.dot(g, wf1g_ref[:], preferred_element_type=jnp.float32)
                    + jnp.dot(obs, wf1o_ref[:], preferred_element_type=jnp.float32)
                    + bf1_ref[:], 0.0)
    c = jnp.maximum(jnp.dot(c, wf2_ref[:], preferred_element_type=jnp.float32)
                    + bf2_ref[:], 0.0)
    out_ref[:] = jnp.tanh(jnp.dot(c, wf3_ref[:], preferred_element_type=jnp.float32)
                          + bf3_ref[:])


def kernel(X, Wg1, bg1, Wg2, bg2, Wp1, bp1, Ws1, Wn1, bc1, Wp2, bp2, Ws2, Wn2,
           bc2, Wf1, bf1, Wf2, bf2, Wf3, bf3):
    batch = X.shape[0]

    grid = (batch // _BB,)
    full = lambda *s: pl.BlockSpec(s, lambda i: (0,) * len(s))

    specs = [
        pl.BlockSpec((_BB, X.shape[1]), lambda i: (i, 0)),
        full(16, _H), full(1, _H),
        full(_H, _H), full(1, _H),
        full(4, 4), full(1, 4),
        full(4, _H), full(4, _H), full(1, _H),
        full(_H, _H), full(1, _H),
        full(_H, _H), full(_H, _H), full(1, _H),
        full(_H, _H), full(_H, _H), full(1, _H),
        full(_H, _H), full(1, _H),
        full(_H, 8), full(1, 8),
    ]
    out = pl.pallas_call(
        _fused_kernel,
        grid=grid,
        in_specs=specs,
        out_specs=pl.BlockSpec((_BB, 8), lambda i: (i, 0)),
        out_shape=jax.ShapeDtypeStruct((batch, 8), jnp.float32),
    )(X,
      Wg1, bg1.reshape(1, -1), Wg2, bg2.reshape(1, -1),
      Wp1, bp1.reshape(1, -1), Ws1, Wn1, bc1.reshape(1, -1),
      Wp2, bp2.reshape(1, -1), Ws2, Wn2, bc2.reshape(1, -1),
      Wf1[:_H], Wf1[_H:], bf1.reshape(1, -1), Wf2, bf2.reshape(1, -1),
      Wf3, bf3.reshape(1, -1))
    return out


# speedup vs baseline: 554.3705x; 1.0070x over previous
import jax
import jax.numpy as jnp
from jax.experimental import pallas as pl

_NOBS, _H, _NPG = 15, 64, 16
_BB = 2048


def _fold_max(a, bb):
    k = a.shape[0] // bb
    while k > 1:
        half = (k // 2) * bb
        a = jnp.maximum(a[:half], a[half:])
        k //= 2
    return a


def _fold_sum(a, bb):
    k = a.shape[0] // bb
    while k > 1:
        half = (k // 2) * bb
        a = a[:half] + a[half:]
        k //= 2
    return a


def _fused_kernel(x_ref,
                  wg1_ref, bg1_ref, wg2_ref, bg2_ref,
                  wp1_ref, bp1_ref, ws1_ref, wn1_ref, bc1_ref,
                  wp2_ref, bp2_ref, ws2_ref, wn2_ref, bc2_ref,
                  wf1g_ref, wf1o_ref, bf1_ref, wf2_ref, bf2_ref,
                  wf3_ref, bf3_ref, out_ref):
    n = _NPG
    bb = x_ref.shape[0]
    x = x_ref[:]

    nd = jnp.concatenate(
        [jnp.zeros((bb, 4), jnp.float32)]
        + [x[:, 12 + 4 * j:16 + 4 * j] for j in range(1, n)], axis=0)

    m1 = jnp.maximum(jnp.dot(nd, wp1_ref[:], preferred_element_type=jnp.float32)
                     + bp1_ref[:], 0.0)
    neigh1 = _fold_max(m1, bb)
    n1w = jnp.dot(neigh1, wn1_ref[:], preferred_element_type=jnp.float32)
    h1 = jnp.maximum(jnp.dot(nd, ws1_ref[:], preferred_element_type=jnp.float32)
                     + jnp.tile(n1w, (n, 1)) + bc1_ref[:], 0.0)

    m2 = jnp.maximum(jnp.dot(h1, wp2_ref[:], preferred_element_type=jnp.float32)
                     + bp2_ref[:], 0.0)
    neigh2 = _fold_max(m2, bb)
    h1mean = _fold_sum(h1, bb) * (1.0 / n)
    obs = (jnp.dot(h1mean, ws2_ref[:], preferred_element_type=jnp.float32)
           + jnp.dot(neigh2, wn2_ref[:], preferred_element_type=jnp.float32)
           + bc2_ref[:])

    g = jnp.maximum(jnp.dot(x[:, :16], wg1_ref[:], preferred_element_type=jnp.float32)
                    + bg1_ref[:], 0.0)
    g = jnp.dot(g, wg2_ref[:], preferred_element_type=jnp.float32) + bg2_ref[:]

    c = jnp.maximum(jnp.dot(g, wf1g_ref[:], preferred_element_type=jnp.float32)
                    + jnp.dot(obs, wf1o_ref[:], preferred_element_type=jnp.float32)
                    + bf1_ref[:], 0.0)
    c = jnp.maximum(jnp.dot(c, wf2_ref[:], preferred_element_type=jnp.float32)
                    + bf2_ref[:], 0.0)
    out_ref[:] = jnp.tanh(jnp.dot(c, wf3_ref[:], preferred_element_type=jnp.float32)
                          + bf3_ref[:])


def kernel(X, Wg1, bg1, Wg2, bg2, Wp1, bp1, Ws1, Wn1, bc1, Wp2, bp2, Ws2, Wn2,
           bc2, Wf1, bf1, Wf2, bf2, Wf3, bf3):
    batch = X.shape[0]

    grid = (batch // _BB,)
    full = lambda *s: pl.BlockSpec(s, lambda i: (0,) * len(s))

    specs = [
        pl.BlockSpec((_BB, X.shape[1]), lambda i: (i, 0)),
        full(16, _H), full(1, _H),
        full(_H, _H), full(1, _H),
        full(4, 4), full(1, 4),
        full(4, _H), full(4, _H), full(1, _H),
        full(_H, _H), full(1, _H),
        full(_H, _H), full(_H, _H), full(1, _H),
        full(_H, _H), full(_H, _H), full(1, _H),
        full(_H, _H), full(1, _H),
        full(_H, 8), full(1, 8),
    ]
    out = pl.pallas_call(
        _fused_kernel,
        grid=grid,
        in_specs=specs,
        out_specs=pl.BlockSpec((_BB, 8), lambda i: (i, 0)),
        out_shape=jax.ShapeDtypeStruct((batch, 8), jnp.float32),
    )(X,
      Wg1, bg1.reshape(1, -1), Wg2, bg2.reshape(1, -1),
      Wp1, bp1.reshape(1, -1), Ws1, Wn1, bc1.reshape(1, -1),
      Wp2, bp2.reshape(1, -1), Ws2, Wn2, bc2.reshape(1, -1),
      Wf1[:_H], Wf1[_H:], bf1.reshape(1, -1), Wf2, bf2.reshape(1, -1),
      Wf3, bf3.reshape(1, -1))
    return out
